# Initial kernel scaffold; baseline (speedup 1.0000x reference)
import jax
import jax.numpy as jnp
from jax.experimental import pallas as pl


def kernel(h, m_ij, x, vel_all, edge_index, W1, b1, W2, b2, W3, b3, W4, b4):
    raise NotImplementedError("write your pallas kernel here")


# speedup vs baseline: 12.6305x; 12.6305x over previous
import functools

import jax
import jax.numpy as jnp
from jax import lax
from jax.experimental import pallas as pl
from jax.experimental.pallas import tpu as pltpu
from jax.experimental.pallas import tpu_sc as plsc

N_NODES = 10000
N_EDGES = 320000
HID = 128

NP = 10240
BE = 2560
NBE = N_EDGES // BE
BN = 2048
NW = 32
EP = N_EDGES // NW
CH = 2000
NCH = EP // CH
NACC = 5


def _edge_mlp_body(m_ref, w1_ref, b1_ref, w2_ref, b2_ref, o_ref):
    tT = lax.dot_general(w1_ref[...], m_ref[...], (((1,), (1,)), ((), ())),
                         preferred_element_type=jnp.float32)
    tT = tT + b1_ref[...]
    midT = tT * jax.nn.sigmoid(tT)
    wrow = lax.dot_general(w2_ref[...], midT, (((1,), (0,)), ((), ())),
                           preferred_element_type=jnp.float32)
    o_ref[...] = jnp.expand_dims(wrow + b2_ref[...], 0)


def _edge_mlp(m_ij, W1, b1, W2, b2):
    return pl.pallas_call(
        _edge_mlp_body,
        grid=(NBE,),
        in_specs=[
            pl.BlockSpec((BE, HID), lambda i: (i, 0)),
            pl.BlockSpec((HID, HID), lambda i: (0, 0)),
            pl.BlockSpec((HID, 1), lambda i: (0, 0)),
            pl.BlockSpec((1, HID), lambda i: (0, 0)),
            pl.BlockSpec((1, 1), lambda i: (0, 0)),
        ],
        out_specs=pl.BlockSpec((1, 1, BE), lambda i: (i, 0, 0)),
        out_shape=jax.ShapeDtypeStruct((NBE, 1, BE), jnp.float32),
    )(m_ij, W1, b1.reshape(HID, 1), W2, b2.reshape(1, 1))


def _make_sc_scatter():
    mesh = plsc.VectorSubcoreMesh(core_axis_name="c", subcore_axis_name="s")

    @functools.partial(
        pl.kernel,
        mesh=mesh,
        compiler_params=pltpu.CompilerParams(needs_layout_passes=False),
        out_type=jax.ShapeDtypeStruct((NW, NACC * NP), jnp.float32),
        scratch_types=[
            pltpu.VMEM((NACC * NP,), jnp.float32),
            pltpu.VMEM((3 * NP,), jnp.float32),
            pltpu.VMEM((CH,), jnp.int32),
            pltpu.VMEM((CH,), jnp.int32),
            pltpu.VMEM((CH,), jnp.float32),
        ],
    )
    def sc_scatter(src_hbm, dst_hbm, w_hbm, xt_hbm, out_hbm,
                   acc, xv, sbuf, dbuf, wbuf):
        wid = lax.axis_index("s") * 2 + lax.axis_index("c")
        base = wid * EP

        def _zero(i, _):
            acc[pl.ds(pl.multiple_of(i * 16, 16), 16)] = jnp.zeros((16,), jnp.float32)
            return _
        lax.fori_loop(0, (NACC * NP) // 16, _zero, None)

        pltpu.sync_copy(xt_hbm, xv)

        ones = jnp.full((16,), 1.0, jnp.float32)
        for j in range(NCH):
            off = base + j * CH
            pltpu.sync_copy(src_hbm.at[pl.ds(off, CH)], sbuf)
            pltpu.sync_copy(dst_hbm.at[pl.ds(off, CH)], dbuf)
            pltpu.sync_copy(w_hbm.at[pl.ds(off, CH)], wbuf)

            def _body(i, _):
                o = pl.ds(pl.multiple_of(i * 16, 16), 16)
                s = sbuf[o]
                d = dbuf[o]
                ww = wbuf[o]
                xs0 = plsc.load_gather(xv, [s])
                xs1 = plsc.load_gather(xv, [s + NP])
                xs2 = plsc.load_gather(xv, [s + 2 * NP])
                plsc.addupdate_scatter(acc, [d], xs0 * ww)
                plsc.addupdate_scatter(acc, [d + NP], xs1 * ww)
                plsc.addupdate_scatter(acc, [d + 2 * NP], xs2 * ww)
                plsc.addupdate_scatter(acc, [d + 3 * NP], ww)
                plsc.addupdate_scatter(acc, [d + 4 * NP], ones)
                return _
            lax.fori_loop(0, CH // 16, _body, None)

        pltpu.sync_copy(acc, out_hbm.at[wid])

    return sc_scatter


_sc_scatter = _make_sc_scatter()


def _combine_body(h_ref, w3_ref, b3_ref, w4_ref, b4_ref,
                  velt_ref, xt_ref, part_ref, o_ref):
    p = part_ref[...]
    accP = p[0]
    for t in range(1, NW):
        accP = accP + p[t]

    wn = accP[3:4, :]
    cnt = accP[4:5, :]
    denom = jnp.maximum(cnt, 1.0)

    gT = lax.dot_general(w3_ref[...], h_ref[...], (((1,), (1,)), ((), ())),
                         preferred_element_type=jnp.float32)
    gT = gT + b3_ref[...]
    gT = gT * jax.nn.sigmoid(gT)
    alphaT = lax.dot_general(w4_ref[...], gT, (((1,), (0,)), ((), ())),
                             preferred_element_type=jnp.float32)
    alphaT = alphaT + b4_ref[...]

    rows = []
    for c in range(3):
        geom = (accP[c:c + 1, :] - xt_ref[c:c + 1, :] * wn) / denom
        vel = alphaT[0:1, :] * velt_ref[c:c + 1, :]
        for j in range(1, 5):
            vel = vel + alphaT[j:j + 1, :] * velt_ref[3 * j + c:3 * j + c + 1, :]
        rows.append(vel + geom)
    o_ref[...] = jnp.concatenate(rows, axis=0)


def _combine(h_p, W3, b3, W4, b4, velt_p, xt_p, partials):
    return pl.pallas_call(
        _combine_body,
        grid=(NP // BN,),
        in_specs=[
            pl.BlockSpec((BN, HID), lambda i: (i, 0)),
            pl.BlockSpec((HID, HID), lambda i: (0, 0)),
            pl.BlockSpec((HID, 1), lambda i: (0, 0)),
            pl.BlockSpec((5, HID), lambda i: (0, 0)),
            pl.BlockSpec((5, 1), lambda i: (0, 0)),
            pl.BlockSpec((15, BN), lambda i: (0, i)),
            pl.BlockSpec((3, BN), lambda i: (0, i)),
            pl.BlockSpec((NW, NACC, BN), lambda i: (0, 0, i)),
        ],
        out_specs=pl.BlockSpec((3, BN), lambda i: (0, i)),
        out_shape=jax.ShapeDtypeStruct((3, NP), jnp.float32),
    )(h_p, W3, b3.reshape(HID, 1), W4, b4.reshape(5, 1), velt_p, xt_p, partials)


def kernel(h, m_ij, x, vel_all, edge_index, W1, b1, W2, b2, W3, b3, W4, b4):
    src = edge_index[0].astype(jnp.int32)
    dst = edge_index[1].astype(jnp.int32)

    w_e = _edge_mlp(m_ij, W1, b1, W2, b2).reshape(N_EDGES)

    xpad = jnp.pad(x, ((0, NP - N_NODES), (0, 0)))
    xt = xpad.T
    xt_flat = xt.reshape(3 * NP)
    partials = _sc_scatter(src, dst, w_e, xt_flat)
    partials = partials.reshape(NW, NACC, NP)

    h_p = jnp.pad(h, ((0, NP - N_NODES), (0, 0)))
    velt = vel_all.reshape(N_NODES, 15).T
    velt_p = jnp.pad(velt, ((0, 0), (0, NP - N_NODES)))
    outT = _combine(h_p, W3, b3, W4, b4, velt_p, xt, partials)
    return outT[:, :N_NODES].T


# speedup vs baseline: 18.1000x; 1.4330x over previous
import functools

import jax
import jax.numpy as jnp
from jax import lax
from jax.experimental import pallas as pl
from jax.experimental.pallas import tpu as pltpu
from jax.experimental.pallas import tpu_sc as plsc

N_NODES = 10000
N_EDGES = 320000
HID = 128

NP = 10240
BE = 6400
NBE = N_EDGES // BE
BN = 2048
NW = 32
EP = N_EDGES // NW
CH = 2000
NCH = EP // CH
NACC = 5


def _edge_mlp_body(m_ref, w1_ref, b1_ref, w2_ref, b2_ref, o_ref):
    tT = lax.dot_general(w1_ref[...], m_ref[...], (((1,), (1,)), ((), ())),
                         preferred_element_type=jnp.float32)
    tT = tT + b1_ref[...]
    midT = tT * jax.nn.sigmoid(tT)
    wrow = lax.dot_general(w2_ref[...], midT, (((1,), (0,)), ((), ())),
                           preferred_element_type=jnp.float32)
    o_ref[...] = jnp.expand_dims(wrow + b2_ref[...], 0)


def _edge_mlp(m_ij, W1, b1, W2, b2):
    return pl.pallas_call(
        _edge_mlp_body,
        grid=(NBE,),
        in_specs=[
            pl.BlockSpec((BE, HID), lambda i: (i, 0)),
            pl.BlockSpec((HID, HID), lambda i: (0, 0)),
            pl.BlockSpec((HID, 1), lambda i: (0, 0)),
            pl.BlockSpec((1, HID), lambda i: (0, 0)),
            pl.BlockSpec((1, 1), lambda i: (0, 0)),
        ],
        out_specs=pl.BlockSpec((1, 1, BE), lambda i: (i, 0, 0)),
        out_shape=jax.ShapeDtypeStruct((NBE, 1, BE), jnp.float32),
    )(m_ij, W1, b1.reshape(HID, 1), W2, b2.reshape(1, 1))


def _make_sc_scatter():
    mesh = plsc.VectorSubcoreMesh(core_axis_name="c", subcore_axis_name="s")

    @functools.partial(
        pl.kernel,
        mesh=mesh,
        compiler_params=pltpu.CompilerParams(needs_layout_passes=False),
        out_type=jax.ShapeDtypeStruct((NW, NACC, NP), jnp.float32),
        scratch_types=[
            pltpu.VMEM((NACC, NP), jnp.float32),
            pltpu.VMEM((3 * NP,), jnp.float32),
            pltpu.VMEM((CH,), jnp.int32),
            pltpu.VMEM((CH,), jnp.int32),
            pltpu.VMEM((CH,), jnp.float32),
            pltpu.VMEM((CH,), jnp.int32),
            pltpu.VMEM((CH,), jnp.int32),
            pltpu.VMEM((CH,), jnp.float32),
            pltpu.SemaphoreType.DMA,
        ],
    )
    def sc_scatter(src_hbm, dst_hbm, w_hbm, xt_hbm, out_hbm,
                   acc, xv, s0, d0, w0, s1, d1, w1, sem):
        wid = lax.axis_index("s") * 2 + lax.axis_index("c")
        base = wid * EP
        bufs = ((s0, d0, w0), (s1, d1, w1))

        def _stage(j, k):
            off = base + j * CH
            sb, db, wb = bufs[k]
            return (pltpu.async_copy(src_hbm.at[pl.ds(off, CH)], sb, sem),
                    pltpu.async_copy(dst_hbm.at[pl.ds(off, CH)], db, sem),
                    pltpu.async_copy(w_hbm.at[pl.ds(off, CH)], wb, sem))

        cx = pltpu.async_copy(xt_hbm, xv, sem)
        pend = _stage(0, 0)

        zrow = jnp.zeros((16,), jnp.float32)

        def _zero(i, _):
            for q in range(NACC):
                acc[q, pl.ds(pl.multiple_of(i * 16, 16), 16)] = zrow
            return _
        lax.fori_loop(0, NP // 16, _zero, None, unroll=4)
        cx.wait()

        ones = jnp.full((16,), 1.0, jnp.float32)
        q0 = jnp.zeros((16,), jnp.int32)
        q1 = jnp.full((16,), 1, jnp.int32)
        q2 = jnp.full((16,), 2, jnp.int32)
        q3 = jnp.full((16,), 3, jnp.int32)
        q4 = jnp.full((16,), 4, jnp.int32)

        for j in range(NCH):
            k = j % 2
            sb, db, wb = bufs[k]
            for c in pend:
                c.wait()
            if j + 1 < NCH:
                pend = _stage(j + 1, 1 - k)

            def _body(i, _):
                o = pl.ds(pl.multiple_of(i * 16, 16), 16)
                s = sb[o]
                d = db[o]
                ww = wb[o]
                xs0 = plsc.load_gather(xv, [s])
                xs1 = plsc.load_gather(xv, [s + NP])
                xs2 = plsc.load_gather(xv, [s + 2 * NP])
                plsc.addupdate_scatter(acc, [q0, d], xs0 * ww)
                plsc.addupdate_scatter(acc, [q1, d], xs1 * ww)
                plsc.addupdate_scatter(acc, [q2, d], xs2 * ww)
                plsc.addupdate_scatter(acc, [q3, d], ww)
                plsc.addupdate_scatter(acc, [q4, d], ones)
                return _
            lax.fori_loop(0, CH // 16, _body, None, unroll=2)

        pltpu.sync_copy(acc, out_hbm.at[wid])

    return sc_scatter


_sc_scatter = _make_sc_scatter()


def _combine_body(h_ref, w3_ref, b3_ref, w4_ref, b4_ref,
                  velt_ref, xt_ref, part_ref, o_ref):
    p = part_ref[...]
    accP = p[0]
    for t in range(1, NW):
        accP = accP + p[t]

    wn = accP[3:4, :]
    cnt = accP[4:5, :]
    denom = jnp.maximum(cnt, 1.0)

    gT = lax.dot_general(w3_ref[...], h_ref[...], (((1,), (1,)), ((), ())),
                         preferred_element_type=jnp.float32)
    gT = gT + b3_ref[...]
    gT = gT * jax.nn.sigmoid(gT)
    alphaT = lax.dot_general(w4_ref[...], gT, (((1,), (0,)), ((), ())),
                             preferred_element_type=jnp.float32)
    alphaT = alphaT + b4_ref[...]

    rows = []
    for c in range(3):
        geom = (accP[c:c + 1, :] - xt_ref[c:c + 1, :] * wn) / denom
        vel = alphaT[0:1, :] * velt_ref[c:c + 1, :]
        for j in range(1, 5):
            vel = vel + alphaT[j:j + 1, :] * velt_ref[3 * j + c:3 * j + c + 1, :]
        rows.append(vel + geom)
    o_ref[...] = jnp.concatenate(rows, axis=0)


def _combine(h_p, W3, b3, W4, b4, velt_p, xt_p, partials):
    return pl.pallas_call(
        _combine_body,
        grid=(NP // BN,),
        in_specs=[
            pl.BlockSpec((BN, HID), lambda i: (i, 0)),
            pl.BlockSpec((HID, HID), lambda i: (0, 0)),
            pl.BlockSpec((HID, 1), lambda i: (0, 0)),
            pl.BlockSpec((5, HID), lambda i: (0, 0)),
            pl.BlockSpec((5, 1), lambda i: (0, 0)),
            pl.BlockSpec((15, BN), lambda i: (0, i)),
            pl.BlockSpec((3, BN), lambda i: (0, i)),
            pl.BlockSpec((NW, NACC, BN), lambda i: (0, 0, i)),
        ],
        out_specs=pl.BlockSpec((3, BN), lambda i: (0, i)),
        out_shape=jax.ShapeDtypeStruct((3, NP), jnp.float32),
    )(h_p, W3, b3.reshape(HID, 1), W4, b4.reshape(5, 1), velt_p, xt_p, partials)


def kernel(h, m_ij, x, vel_all, edge_index, W1, b1, W2, b2, W3, b3, W4, b4):
    src = edge_index[0].astype(jnp.int32)
    dst = edge_index[1].astype(jnp.int32)

    w_e = _edge_mlp(m_ij, W1, b1, W2, b2).reshape(N_EDGES)

    xpad = jnp.pad(x, ((0, NP - N_NODES), (0, 0)))
    xt = xpad.T
    xt_flat = xt.reshape(3 * NP)
    partials = _sc_scatter(src, dst, w_e, xt_flat)

    h_p = jnp.pad(h, ((0, NP - N_NODES), (0, 0)))
    velt = vel_all.reshape(N_NODES, 15).T
    velt_p = jnp.pad(velt, ((0, 0), (0, NP - N_NODES)))
    outT = _combine(h_p, W3, b3, W4, b4, velt_p, xt, partials)
    return outT[:, :N_NODES].T


# speedup vs baseline: 19.0768x; 1.0540x over previous
import functools

import jax
import jax.numpy as jnp
from jax import lax
from jax.experimental import pallas as pl
from jax.experimental.pallas import tpu as pltpu
from jax.experimental.pallas import tpu_sc as plsc

N_NODES = 10000
N_EDGES = 320000
HID = 128

NP = 10240
BE = 6400
NBE = N_EDGES // BE
BN = 2048
NW = 32
EP = N_EDGES // NW
CH = 2000
NCH = EP // CH
NACC = 5


def _edge_mlp_body(m_ref, w1_ref, b1_ref, w2_ref, b2_ref, o_ref):
    m16 = m_ref[...].astype(jnp.bfloat16)
    w116 = w1_ref[...].astype(jnp.bfloat16)
    tT = lax.dot_general(w116, m16, (((1,), (1,)), ((), ())),
                         preferred_element_type=jnp.float32)
    tT = tT + b1_ref[...]
    midT = tT * (0.5 * jnp.tanh(tT * 0.5) + 0.5)
    wrow = lax.dot_general(w2_ref[...], midT, (((1,), (0,)), ((), ())),
                           preferred_element_type=jnp.float32)
    o_ref[...] = jnp.expand_dims(wrow + b2_ref[...], 0)


def _edge_mlp(m_ij, W1, b1, W2, b2):
    return pl.pallas_call(
        _edge_mlp_body,
        grid=(NBE,),
        in_specs=[
            pl.BlockSpec((BE, HID), lambda i: (i, 0)),
            pl.BlockSpec((HID, HID), lambda i: (0, 0)),
            pl.BlockSpec((HID, 1), lambda i: (0, 0)),
            pl.BlockSpec((1, HID), lambda i: (0, 0)),
            pl.BlockSpec((1, 1), lambda i: (0, 0)),
        ],
        out_specs=pl.BlockSpec((1, 1, BE), lambda i: (i, 0, 0)),
        out_shape=jax.ShapeDtypeStruct((NBE, 1, BE), jnp.float32),
    )(m_ij, W1, b1.reshape(HID, 1), W2, b2.reshape(1, 1))


def _make_sc_scatter():
    mesh = plsc.VectorSubcoreMesh(core_axis_name="c", subcore_axis_name="s")

    @functools.partial(
        pl.kernel,
        mesh=mesh,
        compiler_params=pltpu.CompilerParams(needs_layout_passes=False),
        out_type=jax.ShapeDtypeStruct((NW, NACC, NP), jnp.float32),
        scratch_types=[
            pltpu.VMEM((NACC, NP), jnp.float32),
            pltpu.VMEM((3 * NP,), jnp.float32),
            pltpu.VMEM((CH,), jnp.int32),
            pltpu.VMEM((CH,), jnp.int32),
            pltpu.VMEM((CH,), jnp.float32),
            pltpu.VMEM((CH,), jnp.int32),
            pltpu.VMEM((CH,), jnp.int32),
            pltpu.VMEM((CH,), jnp.float32),
            pltpu.SemaphoreType.DMA,
        ],
    )
    def sc_scatter(src_hbm, dst_hbm, w_hbm, xt_hbm, out_hbm,
                   acc, xv, s0, d0, w0, s1, d1, w1, sem):
        wid = lax.axis_index("s") * 2 + lax.axis_index("c")
        base = wid * EP
        bufs = ((s0, d0, w0), (s1, d1, w1))

        def _stage(j, k):
            off = base + j * CH
            sb, db, wb = bufs[k]
            return (pltpu.async_copy(src_hbm.at[pl.ds(off, CH)], sb, sem),
                    pltpu.async_copy(dst_hbm.at[pl.ds(off, CH)], db, sem),
                    pltpu.async_copy(w_hbm.at[pl.ds(off, CH)], wb, sem))

        cx = pltpu.async_copy(xt_hbm, xv, sem)
        pend = _stage(0, 0)

        zrow = jnp.zeros((16,), jnp.float32)

        def _zero(i, _):
            for q in range(NACC):
                acc[q, pl.ds(pl.multiple_of(i * 16, 16), 16)] = zrow
            return _
        lax.fori_loop(0, NP // 16, _zero, None, unroll=4)
        cx.wait()

        ones = jnp.full((16,), 1.0, jnp.float32)
        q0 = jnp.zeros((16,), jnp.int32)
        q1 = jnp.full((16,), 1, jnp.int32)
        q2 = jnp.full((16,), 2, jnp.int32)
        q3 = jnp.full((16,), 3, jnp.int32)
        q4 = jnp.full((16,), 4, jnp.int32)

        for j in range(NCH):
            k = j % 2
            sb, db, wb = bufs[k]
            for c in pend:
                c.wait()
            if j + 1 < NCH:
                pend = _stage(j + 1, 1 - k)

            def _body(i, _):
                o = pl.ds(pl.multiple_of(i * 16, 16), 16)
                s = sb[o]
                d = db[o]
                ww = wb[o]
                xs0 = plsc.load_gather(xv, [s])
                xs1 = plsc.load_gather(xv, [s + NP])
                xs2 = plsc.load_gather(xv, [s + 2 * NP])
                plsc.addupdate_scatter(acc, [q0, d], xs0 * ww)
                plsc.addupdate_scatter(acc, [q1, d], xs1 * ww)
                plsc.addupdate_scatter(acc, [q2, d], xs2 * ww)
                plsc.addupdate_scatter(acc, [q3, d], ww)
                plsc.addupdate_scatter(acc, [q4, d], ones)
                return _
            lax.fori_loop(0, CH // 16, _body, None, unroll=2)

        pltpu.sync_copy(acc, out_hbm.at[wid])

    return sc_scatter


_sc_scatter = _make_sc_scatter()


def _node_mlp_body(h_ref, w3_ref, b3_ref, w4_ref, b4_ref, velt_ref, o_ref):
    gT = lax.dot_general(w3_ref[...], h_ref[...], (((1,), (1,)), ((), ())),
                         preferred_element_type=jnp.float32)
    gT = gT + b3_ref[...]
    gT = gT * jax.nn.sigmoid(gT)
    alphaT = lax.dot_general(w4_ref[...], gT, (((1,), (0,)), ((), ())),
                             preferred_element_type=jnp.float32)
    alphaT = alphaT + b4_ref[...]

    rows = []
    for c in range(3):
        vel = alphaT[0:1, :] * velt_ref[c:c + 1, :]
        for j in range(1, 5):
            vel = vel + alphaT[j:j + 1, :] * velt_ref[3 * j + c:3 * j + c + 1, :]
        rows.append(vel)
    o_ref[...] = jnp.concatenate(rows, axis=0)


def _node_mlp(h_p, W3, b3, W4, b4, velt_p):
    return pl.pallas_call(
        _node_mlp_body,
        grid=(NP // BN,),
        in_specs=[
            pl.BlockSpec((BN, HID), lambda i: (i, 0)),
            pl.BlockSpec((HID, HID), lambda i: (0, 0)),
            pl.BlockSpec((HID, 1), lambda i: (0, 0)),
            pl.BlockSpec((5, HID), lambda i: (0, 0)),
            pl.BlockSpec((5, 1), lambda i: (0, 0)),
            pl.BlockSpec((15, BN), lambda i: (0, i)),
        ],
        out_specs=pl.BlockSpec((3, BN), lambda i: (0, i)),
        out_shape=jax.ShapeDtypeStruct((3, NP), jnp.float32),
    )(h_p, W3, b3.reshape(HID, 1), W4, b4.reshape(5, 1), velt_p)


def _combine_body(velc_ref, xt_ref, part_ref, o_ref):
    p = part_ref[...]
    accP = p[0]
    for t in range(1, NW):
        accP = accP + p[t]

    wn = accP[3:4, :]
    cnt = accP[4:5, :]
    denom = jnp.maximum(cnt, 1.0)

    rows = []
    for c in range(3):
        geom = (accP[c:c + 1, :] - xt_ref[c:c + 1, :] * wn) / denom
        rows.append(velc_ref[c:c + 1, :] + geom)
    o_ref[...] = jnp.concatenate(rows, axis=0)


def _combine(velc, xt_p, partials):
    return pl.pallas_call(
        _combine_body,
        grid=(NP // BN,),
        in_specs=[
            pl.BlockSpec((3, BN), lambda i: (0, i)),
            pl.BlockSpec((3, BN), lambda i: (0, i)),
            pl.BlockSpec((NW, NACC, BN), lambda i: (0, 0, i)),
        ],
        out_specs=pl.BlockSpec((3, BN), lambda i: (0, i)),
        out_shape=jax.ShapeDtypeStruct((3, NP), jnp.float32),
    )(velc, xt_p, partials)


def kernel(h, m_ij, x, vel_all, edge_index, W1, b1, W2, b2, W3, b3, W4, b4):
    src = edge_index[0].astype(jnp.int32)
    dst = edge_index[1].astype(jnp.int32)

    w_e = _edge_mlp(m_ij, W1, b1, W2, b2).reshape(N_EDGES)

    xpad = jnp.pad(x, ((0, NP - N_NODES), (0, 0)))
    xt = xpad.T
    xt_flat = xt.reshape(3 * NP)
    partials = _sc_scatter(src, dst, w_e, xt_flat)

    h_p = jnp.pad(h, ((0, NP - N_NODES), (0, 0)))
    velt = vel_all.reshape(N_NODES, 15).T
    velt_p = jnp.pad(velt, ((0, 0), (0, NP - N_NODES)))
    velc = _node_mlp(h_p, W3, b3, W4, b4, velt_p)

    outT = _combine(velc, xt, partials)
    return outT[:, :N_NODES].T
